# baseline (speedup 1.0000x reference)
import functools

import jax
import jax.numpy as jnp
from jax import lax
from jax.experimental import pallas as pl
from jax.experimental.pallas import tpu as pltpu
from jax.experimental.pallas import tpu_sc as plsc

N = 10000
E = 320000
ETOT = E + N
NC = 2
NS = 16
NW = NC * NS
K = 128
G = 8
S = G * K
NG = 11
EW = NG * S
EPAD = EW * NW
RB = 1000
GRID = N // RB
NP = 10240

NUM_CLASSES_OUT = 64

_f32 = jnp.float32
_i32 = jnp.int32


def _tc_first(x, W, A2):
    def body(x_ref, w_ref, a_ref, h_ref, als_ref, ald_ref):
        h = jnp.dot(x_ref[...], w_ref[...], preferred_element_type=_f32)
        h_ref[...] = h
        al = jnp.dot(h, a_ref[...], preferred_element_type=_f32)
        als_ref[...] = al[:, 0:1]
        ald_ref[...] = al[:, 1:2]

    do = W.shape[1]
    return pl.pallas_call(
        body,
        grid=(GRID,),
        in_specs=[
            pl.BlockSpec((RB, 128), lambda i: (i, 0)),
            pl.BlockSpec((128, do), lambda i: (0, 0)),
            pl.BlockSpec((do, 2), lambda i: (0, 0)),
        ],
        out_specs=[
            pl.BlockSpec((RB, do), lambda i: (i, 0)),
            pl.BlockSpec((RB, 1), lambda i: (i, 0)),
            pl.BlockSpec((RB, 1), lambda i: (i, 0)),
        ],
        out_shape=[
            jax.ShapeDtypeStruct((N, do), _f32),
            jax.ShapeDtypeStruct((N, 1), _f32),
            jax.ShapeDtypeStruct((N, 1), _f32),
        ],
    )(x, W, A2)


def _tc_mid(acc, den, b, W, A2):
    di = acc.shape[2]
    do = W.shape[1]

    def body(a0_ref, a1_ref, d0_ref, d1_ref, b_ref, w_ref, a_ref,
             h_ref, als_ref, ald_ref):
        s = a0_ref[...] + a1_ref[...]
        d = d0_ref[...] + d1_ref[...] + 1e-16
        x = jnp.maximum(s / d + b_ref[...], 0.0)
        h = jnp.dot(x, w_ref[...], preferred_element_type=_f32)
        h_ref[...] = h
        al = jnp.dot(h, a_ref[...], preferred_element_type=_f32)
        als_ref[...] = al[:, 0:1]
        ald_ref[...] = al[:, 1:2]

    den3 = den.reshape(NC, NP, 1)
    return pl.pallas_call(
        body,
        grid=(GRID,),
        in_specs=[
            pl.BlockSpec((RB, di), lambda i: (i, 0)),
            pl.BlockSpec((RB, di), lambda i: (i, 0)),
            pl.BlockSpec((RB, 1), lambda i: (i, 0)),
            pl.BlockSpec((RB, 1), lambda i: (i, 0)),
            pl.BlockSpec((1, di), lambda i: (0, 0)),
            pl.BlockSpec((di, do), lambda i: (0, 0)),
            pl.BlockSpec((do, 2), lambda i: (0, 0)),
        ],
        out_specs=[
            pl.BlockSpec((RB, do), lambda i: (i, 0)),
            pl.BlockSpec((RB, 1), lambda i: (i, 0)),
            pl.BlockSpec((RB, 1), lambda i: (i, 0)),
        ],
        out_shape=[
            jax.ShapeDtypeStruct((N, do), _f32),
            jax.ShapeDtypeStruct((N, 1), _f32),
            jax.ShapeDtypeStruct((N, 1), _f32),
        ],
    )(acc[0], acc[1], den3[0], den3[1], b.reshape(1, di), W, A2)


def _tc_final(acc, den, b):
    di = acc.shape[2]

    def body(a0_ref, a1_ref, d0_ref, d1_ref, b_ref, o_ref):
        s = a0_ref[...] + a1_ref[...]
        d = d0_ref[...] + d1_ref[...] + 1e-16
        o_ref[...] = s / d + b_ref[...]

    den3 = den.reshape(NC, NP, 1)
    return pl.pallas_call(
        body,
        grid=(GRID,),
        in_specs=[
            pl.BlockSpec((RB, di), lambda i: (i, 0)),
            pl.BlockSpec((RB, di), lambda i: (i, 0)),
            pl.BlockSpec((RB, 1), lambda i: (i, 0)),
            pl.BlockSpec((RB, 1), lambda i: (i, 0)),
            pl.BlockSpec((1, di), lambda i: (0, 0)),
        ],
        out_specs=pl.BlockSpec((RB, di), lambda i: (i, 0)),
        out_shape=jax.ShapeDtypeStruct((N, di), _f32),
    )(acc[0], acc[1], den3[0], den3[1], b.reshape(1, di))


def _make_sc(H):
    HG = H // 16
    mesh = plsc.VectorSubcoreMesh(core_axis_name="c", subcore_axis_name="s")

    @functools.partial(
        pl.kernel,
        out_type=[
            jax.ShapeDtypeStruct((NC, NP, H), _f32),
            jax.ShapeDtypeStruct((NC, NP), _f32),
        ],
        mesh=mesh,
        compiler_params=pltpu.CompilerParams(needs_layout_passes=False),
        scratch_types=[
            pltpu.VMEM((S,), _i32),
            pltpu.VMEM((S,), _i32),
            pltpu.VMEM((S,), _f32),
            pltpu.VMEM((S,), _f32),
            pltpu.VMEM((S,), _f32),
            pltpu.VMEM((K, H), _f32),
            pltpu.VMEM((K, H), _f32),
            pltpu.VMEM((128,), _f32),
            pltpu.VMEM_SHARED((NP,), _f32),
            pltpu.VMEM_SHARED((NP,), _f32),
            pltpu.VMEM_SHARED((NP, H), _f32),
            pltpu.VMEM_SHARED((NP,), _f32),
            pltpu.SemaphoreType.DMA,
            pltpu.SemaphoreType.DMA,
            pltpu.SemaphoreType.DMA,
            pltpu.SemaphoreType.DMA,
            pltpu.SemaphoreType.DMA,
            pltpu.SemaphoreType.DMA,
            pltpu.SemaphoreType.DMA,
        ],
    )
    def sck(h_hbm, als_hbm, ald_hbm, src_hbm, dstf_hbm,
            acc_out, den_out, src_g, dst_f, als_g, ald_g, ex_g,
            rows_a, rows_b, zd_v, als_sh, ald_sh, acc_sh, den_sh,
            g_sa, g_sb, g_la, g_lb, s_sa, s_sb, d_s):
        c = lax.axis_index("c")
        s = lax.axis_index("s")
        wid = s * NC + c
        iota = lax.iota(_i32, 16)
        zeros_f = jnp.zeros((16,), _f32)

        def zrow_body(j, carry):
            for r in range(HG):
                rows_a[j, pl.ds(r * 16, 16)] = zeros_f
            return carry
        lax.fori_loop(0, K, zrow_body, 0)
        def zd_body(g, carry):
            zd_v[pl.ds(g * 16, 16)] = zeros_f
            return carry
        lax.fori_loop(0, 8, zd_body, 0)

        rbase = s * 640

        def zacc_body(k, carry):
            pltpu.sync_copy(rows_a.at[pl.ds(0, K)],
                            acc_sh.at[pl.ds(rbase + k * K, K)])
            return carry
        lax.fori_loop(0, 5, zacc_body, 0)

        def zden_body(k, carry):
            pltpu.sync_copy(zd_v, den_sh.at[pl.ds(rbase + k * 128, 128)])
            return carry
        lax.fori_loop(0, 5, zden_body, 0)

        @pl.when(s == 0)
        def _():
            pltpu.sync_copy(als_hbm, als_sh)
            pltpu.sync_copy(ald_hbm, ald_sh)

        ebase = wid * EW
        plsc.subcore_barrier()

        def scale_rows(rows_v, exbase):
            def grp_body(g, icarry):
                gbase = pl.multiple_of(g * 16, 16)
                exg = ex_g[pl.ds(exbase + gbase, 16)]
                for jj in range(16):
                    a = exg[jj]
                    j = gbase + jj
                    for r in range(HG):
                        sl = pl.ds(r * 16, 16)
                        rows_v[j, sl] = rows_v[j, sl] * a
                return icarry
            lax.fori_loop(0, K // 16, grp_body, 0)

        def group_body(it, carry):
            gb = pl.multiple_of(it * S, S)
            pltpu.sync_copy(src_hbm.at[pl.ds(ebase + gb, S)], src_g)
            pltpu.sync_copy(dstf_hbm.at[pl.ds(ebase + gb, S)], dst_f)
            cl_a = pltpu.async_copy(als_sh.at[src_g], als_g, g_la)
            cl_b = pltpu.async_copy(ald_sh.at[dst_f], ald_g, g_lb)
            rows = (rows_a, rows_b)
            gsem = (g_sa, g_sb)
            ssem = (s_sa, s_sb)
            gd = [pltpu.async_copy(h_hbm.at[src_g.at[pl.ds(0, K)]],
                                   rows_a, g_sa), None]
            cl_a.wait()
            cl_b.wait()
            def ex_body(g, icarry):
                sl = pl.ds(g * 16, 16)
                e = als_g[sl] + ald_g[sl]
                e = jnp.where(e >= 0.0, e, 0.2 * e)
                gid = ebase + gb + g * 16 + iota
                ex = jnp.where(gid < ETOT, jnp.exp(e), 0.0)
                ex_g[sl] = ex
                return icarry
            lax.fori_loop(0, S // 16, ex_body, 0)
            cd = pltpu.async_copy(ex_g, den_sh.at[dst_f], d_s, add=True)
            sd = [None, None]
            for k in range(G):
                p = k & 1
                q = 1 - p
                if k < G - 1:
                    if sd[q] is not None:
                        sd[q].wait()
                    gd[q] = pltpu.async_copy(
                        h_hbm.at[src_g.at[pl.ds((k + 1) * K, K)]],
                        rows[q], gsem[q])
                gd[p].wait()
                scale_rows(rows[p], k * K)
                sd[p] = pltpu.async_copy(rows[p],
                                         acc_sh.at[dst_f.at[pl.ds(k * K, K)]],
                                         ssem[p], add=True)
            sd[0].wait()
            sd[1].wait()
            cd.wait()
            return carry

        lax.fori_loop(0, NG, group_body, 0)

        plsc.subcore_barrier()

        pltpu.sync_copy(acc_sh.at[pl.ds(rbase, 640)],
                        acc_out.at[c, pl.ds(rbase, 640)])
        pltpu.sync_copy(den_sh.at[pl.ds(rbase, 640)],
                        den_out.at[c, pl.ds(rbase, 640)])

    return sck


_SC_CACHE = {}


def _sc_layer(h, als, ald, srcf, dstf):
    H = h.shape[1]
    if H not in _SC_CACHE:
        _SC_CACHE[H] = _make_sc(H)
    als_p = jnp.pad(als.reshape(N), (0, NP - N))
    ald_p = jnp.pad(ald.reshape(N), (0, NP - N))
    return _SC_CACHE[H](h, als_p, ald_p, srcf, dstf)


def kernel(x, edge_index, W1, as1, ad1, b1, W2, as2, ad2, b2,
           W3, as3, ad3, b3, W4, as4, ad4, b4):
    loops = jnp.arange(N, dtype=edge_index.dtype)
    ei = jnp.concatenate([edge_index, jnp.stack([loops, loops])], axis=1)
    pad = jnp.zeros((2, EPAD - ETOT), dtype=ei.dtype)
    ei = jnp.concatenate([ei, pad], axis=1).astype(_i32)
    srcf = ei[0]
    dstf = ei[1]

    W4p = jnp.pad(W4, ((0, 0), (0, 128 - W4.shape[1])))
    A24 = jnp.pad(jnp.stack([as4, ad4], axis=1), ((0, 128 - as4.shape[0]), (0, 0)))
    A = [None,
         (W1, jnp.stack([as1, ad1], axis=1), b1),
         (W2, jnp.stack([as2, ad2], axis=1), b2),
         (W3, jnp.stack([as3, ad3], axis=1), b3),
         (W4p, A24, b4)]

    h, als, ald = _tc_first(x, A[1][0], A[1][1])
    acc, den = _sc_layer(h, als, ald, srcf, dstf)
    for i in (2, 3, 4):
        h, als, ald = _tc_mid(acc, den, A[i - 1][2], A[i][0], A[i][1])
        acc, den = _sc_layer(h, als, ald, srcf, dstf)
    b4p = jnp.pad(A[4][2], (0, 128 - A[4][2].shape[0]))
    return _tc_final(acc, den, b4p)[:, :NUM_CLASSES_OUT]


# speedup vs baseline: 5.9829x; 5.9829x over previous
import functools

import jax
import jax.numpy as jnp
from jax import lax
from jax.experimental import pallas as pl
from jax.experimental.pallas import tpu as pltpu
from jax.experimental.pallas import tpu_sc as plsc

N = 10000
E = 320000
ETOT = E + N
NC = 2
NS = 16
NW = NC * NS
K = 128
G = 8
S = G * K
NG = 11
EW = NG * S
EPAD = EW * NW
RB = 1000
GRID = N // RB
NP = 10240

NUM_CLASSES_OUT = 64

_f32 = jnp.float32
_i32 = jnp.int32


def _tc_first(x, W, A2):
    def body(x_ref, w_ref, a_ref, h_ref, als_ref, ald_ref):
        h = jnp.dot(x_ref[...], w_ref[...], preferred_element_type=_f32)
        h_ref[...] = h
        al = jnp.dot(h, a_ref[...], preferred_element_type=_f32)
        als_ref[...] = al[:, 0:1]
        ald_ref[...] = al[:, 1:2]

    do = W.shape[1]
    return pl.pallas_call(
        body,
        grid=(GRID,),
        in_specs=[
            pl.BlockSpec((RB, 128), lambda i: (i, 0)),
            pl.BlockSpec((128, do), lambda i: (0, 0)),
            pl.BlockSpec((do, 2), lambda i: (0, 0)),
        ],
        out_specs=[
            pl.BlockSpec((RB, do), lambda i: (i, 0)),
            pl.BlockSpec((RB, 1), lambda i: (i, 0)),
            pl.BlockSpec((RB, 1), lambda i: (i, 0)),
        ],
        out_shape=[
            jax.ShapeDtypeStruct((N, do), _f32),
            jax.ShapeDtypeStruct((N, 1), _f32),
            jax.ShapeDtypeStruct((N, 1), _f32),
        ],
    )(x, W, A2)


def _tc_mid(acc, den, b, W, A2):
    di = acc.shape[2]
    do = W.shape[1]

    def body(a0_ref, a1_ref, d0_ref, d1_ref, b_ref, w_ref, a_ref,
             h_ref, als_ref, ald_ref):
        s = a0_ref[...] + a1_ref[...]
        d = d0_ref[...] + d1_ref[...] + 1e-16
        x = jnp.maximum(s / d + b_ref[...], 0.0)
        h = jnp.dot(x, w_ref[...], preferred_element_type=_f32)
        h_ref[...] = h
        al = jnp.dot(h, a_ref[...], preferred_element_type=_f32)
        als_ref[...] = al[:, 0:1]
        ald_ref[...] = al[:, 1:2]

    den3 = den.reshape(NC, NP, 1)
    return pl.pallas_call(
        body,
        grid=(GRID,),
        in_specs=[
            pl.BlockSpec((RB, di), lambda i: (i, 0)),
            pl.BlockSpec((RB, di), lambda i: (i, 0)),
            pl.BlockSpec((RB, 1), lambda i: (i, 0)),
            pl.BlockSpec((RB, 1), lambda i: (i, 0)),
            pl.BlockSpec((1, di), lambda i: (0, 0)),
            pl.BlockSpec((di, do), lambda i: (0, 0)),
            pl.BlockSpec((do, 2), lambda i: (0, 0)),
        ],
        out_specs=[
            pl.BlockSpec((RB, do), lambda i: (i, 0)),
            pl.BlockSpec((RB, 1), lambda i: (i, 0)),
            pl.BlockSpec((RB, 1), lambda i: (i, 0)),
        ],
        out_shape=[
            jax.ShapeDtypeStruct((N, do), _f32),
            jax.ShapeDtypeStruct((N, 1), _f32),
            jax.ShapeDtypeStruct((N, 1), _f32),
        ],
    )(acc[0], acc[1], den3[0], den3[1], b.reshape(1, di), W, A2)


def _tc_final(acc, den, b):
    di = acc.shape[2]

    def body(a0_ref, a1_ref, d0_ref, d1_ref, b_ref, o_ref):
        s = a0_ref[...] + a1_ref[...]
        d = d0_ref[...] + d1_ref[...] + 1e-16
        o_ref[...] = s / d + b_ref[...]

    den3 = den.reshape(NC, NP, 1)
    return pl.pallas_call(
        body,
        grid=(GRID,),
        in_specs=[
            pl.BlockSpec((RB, di), lambda i: (i, 0)),
            pl.BlockSpec((RB, di), lambda i: (i, 0)),
            pl.BlockSpec((RB, 1), lambda i: (i, 0)),
            pl.BlockSpec((RB, 1), lambda i: (i, 0)),
            pl.BlockSpec((1, di), lambda i: (0, 0)),
        ],
        out_specs=pl.BlockSpec((RB, di), lambda i: (i, 0)),
        out_shape=jax.ShapeDtypeStruct((N, di), _f32),
    )(acc[0], acc[1], den3[0], den3[1], b.reshape(1, di))


def _make_sc(H):
    HG = H // 16
    mesh = plsc.VectorSubcoreMesh(core_axis_name="c", subcore_axis_name="s")

    @functools.partial(
        pl.kernel,
        out_type=[
            jax.ShapeDtypeStruct((NC, NP, H), _f32),
            jax.ShapeDtypeStruct((NC, NP), _f32),
        ],
        mesh=mesh,
        compiler_params=pltpu.CompilerParams(needs_layout_passes=False),
        scratch_types=[
            pltpu.VMEM((S,), _i32),
            pltpu.VMEM((S,), _i32),
            pltpu.VMEM((S,), _f32),
            pltpu.VMEM((S,), _f32),
            pltpu.VMEM((S,), _f32),
            pltpu.VMEM((K, H), _f32),
            pltpu.VMEM((K, H), _f32),
            pltpu.VMEM((128,), _f32),
            pltpu.VMEM_SHARED((NP,), _f32),
            pltpu.VMEM_SHARED((NP,), _f32),
            pltpu.VMEM_SHARED((NP, H), _f32),
            pltpu.VMEM_SHARED((NP,), _f32),
            pltpu.SemaphoreType.DMA,
            pltpu.SemaphoreType.DMA,
            pltpu.SemaphoreType.DMA,
            pltpu.SemaphoreType.DMA,
            pltpu.SemaphoreType.DMA,
            pltpu.SemaphoreType.DMA,
            pltpu.SemaphoreType.DMA,
        ],
    )
    def sck(h_hbm, als_hbm, ald_hbm, src_hbm, dstf_hbm,
            acc_out, den_out, src_g, dst_f, als_g, ald_g, ex_g,
            rows_a, rows_b, zd_v, als_sh, ald_sh, acc_sh, den_sh,
            g_sa, g_sb, g_la, g_lb, s_sa, s_sb, d_s):
        c = lax.axis_index("c")
        s = lax.axis_index("s")
        wid = s * NC + c
        iota = lax.iota(_i32, 16)
        zeros_f = jnp.zeros((16,), _f32)

        def zrow_body(j, carry):
            for r in range(HG):
                rows_a[j, pl.ds(r * 16, 16)] = zeros_f
            return carry
        lax.fori_loop(0, K, zrow_body, 0)
        def zd_body(g, carry):
            zd_v[pl.ds(g * 16, 16)] = zeros_f
            return carry
        lax.fori_loop(0, 8, zd_body, 0)

        rbase = s * 640

        def zacc_body(k, carry):
            pltpu.sync_copy(rows_a.at[pl.ds(0, K)],
                            acc_sh.at[pl.ds(rbase + k * K, K)])
            return carry
        lax.fori_loop(0, 5, zacc_body, 0)

        def zden_body(k, carry):
            pltpu.sync_copy(zd_v, den_sh.at[pl.ds(rbase + k * 128, 128)])
            return carry
        lax.fori_loop(0, 5, zden_body, 0)

        @pl.when(s == 0)
        def _():
            pltpu.sync_copy(als_hbm, als_sh)
            pltpu.sync_copy(ald_hbm, ald_sh)

        ebase = wid * EW
        plsc.subcore_barrier()

        def scale_rows(rows_v, exbase):
            def grp_body(g, icarry):
                gbase = pl.multiple_of(g * 16, 16)
                exg = ex_g[pl.ds(exbase + gbase, 16)]
                for jj in range(16):
                    a = exg[jj]
                    j = gbase + jj
                    for r in range(HG):
                        sl = pl.ds(r * 16, 16)
                        rows_v[j, sl] = rows_v[j, sl] * a
                return icarry
            lax.fori_loop(0, K // 16, grp_body, 0)

        def group_body(it, carry):
            gb = pl.multiple_of(it * S, S)
            pltpu.sync_copy(src_hbm.at[pl.ds(ebase + gb, S)], src_g)
            pltpu.sync_copy(dstf_hbm.at[pl.ds(ebase + gb, S)], dst_f)
            cl_a = pltpu.async_copy(als_sh.at[src_g], als_g, g_la)
            cl_b = pltpu.async_copy(ald_sh.at[dst_f], ald_g, g_lb)
            rows = (rows_a, rows_b)
            gsem = (g_sa, g_sb)
            ssem = (s_sa, s_sb)
            gd = [pltpu.async_copy(h_hbm.at[src_g.at[pl.ds(0, K)]],
                                   rows_a, g_sa), None]
            cl_a.wait()
            cl_b.wait()
            def ex_body(g, icarry):
                sl = pl.ds(g * 16, 16)
                e = als_g[sl] + ald_g[sl]
                e = jnp.where(e >= 0.0, e, 0.2 * e)
                gid = ebase + gb + g * 16 + iota
                ex = jnp.where(gid < ETOT, jnp.exp(e), 0.0)
                ex_g[sl] = ex
                return icarry
            lax.fori_loop(0, S // 16, ex_body, 0)
            cd = pltpu.async_copy(ex_g, den_sh.at[dst_f], d_s, add=True)
            sd = [None, None]
            for k in range(G):
                p = k & 1
                q = 1 - p
                if k < G - 1:
                    if sd[q] is not None:
                        sd[q].wait()
                    gd[q] = pltpu.async_copy(
                        h_hbm.at[src_g.at[pl.ds((k + 1) * K, K)]],
                        rows[q], gsem[q])
                gd[p].wait()
                scale_rows(rows[p], k * K)
                sd[p] = pltpu.async_copy(rows[p],
                                         acc_sh.at[dst_f.at[pl.ds(k * K, K)]],
                                         ssem[p], add=True)
            sd[0].wait()
            sd[1].wait()
            cd.wait()
            return carry

        lax.fori_loop(0, NG, group_body, 0)

        plsc.subcore_barrier()

        pltpu.sync_copy(acc_sh.at[pl.ds(rbase, 640)],
                        acc_out.at[c, pl.ds(rbase, 640)])
        pltpu.sync_copy(den_sh.at[pl.ds(rbase, 640)],
                        den_out.at[c, pl.ds(rbase, 640)])

    return sck


_SC_CACHE = {}


def _sc_layer(h, als, ald, srcf, dstf):
    H = h.shape[1]
    if H not in _SC_CACHE:
        _SC_CACHE[H] = _make_sc(H)
    als_p = jnp.pad(als.reshape(N), (0, NP - N))
    ald_p = jnp.pad(ald.reshape(N), (0, NP - N))
    return _SC_CACHE[H](h, als_p, ald_p, srcf, dstf)


def kernel(x, edge_index, W1, as1, ad1, b1, W2, as2, ad2, b2,
           W3, as3, ad3, b3, W4, as4, ad4, b4):
    loops = jnp.arange(N, dtype=edge_index.dtype)
    ei = jnp.concatenate([edge_index, jnp.stack([loops, loops])], axis=1)
    padv = jnp.arange(EPAD - ETOT, dtype=ei.dtype) % N
    pad = jnp.stack([padv, padv])
    ei = jnp.concatenate([ei, pad], axis=1).astype(_i32)
    srcf = ei[0]
    dstf = ei[1]

    W4p = jnp.pad(W4, ((0, 0), (0, 128 - W4.shape[1])))
    A24 = jnp.pad(jnp.stack([as4, ad4], axis=1), ((0, 128 - as4.shape[0]), (0, 0)))
    A = [None,
         (W1, jnp.stack([as1, ad1], axis=1), b1),
         (W2, jnp.stack([as2, ad2], axis=1), b2),
         (W3, jnp.stack([as3, ad3], axis=1), b3),
         (W4p, A24, b4)]

    h, als, ald = _tc_first(x, A[1][0], A[1][1])
    acc, den = _sc_layer(h, als, ald, srcf, dstf)
    for i in (2, 3, 4):
        h, als, ald = _tc_mid(acc, den, A[i - 1][2], A[i][0], A[i][1])
        acc, den = _sc_layer(h, als, ald, srcf, dstf)
    b4p = jnp.pad(A[4][2], (0, 128 - A[4][2].shape[0]))
    return _tc_final(acc, den, b4p)[:, :NUM_CLASSES_OUT]
